# baseline (speedup 1.0000x reference)
import functools

import jax
import jax.numpy as jnp
from jax import lax
from jax.experimental import pallas as pl
from jax.experimental.pallas import tpu as pltpu
from jax.experimental.pallas import tpu_sc as plsc

_CHUNK = 384


def kernel(embed_user, embed_item):
    n_u, e = embed_user.shape
    n_i, _ = embed_item.shape
    info = plsc.get_sparse_core_info()
    nw = info.num_cores * info.num_subcores
    assert n_u == n_i
    n = n_u
    rows_main = -(-n // nw)
    rows_main += (-rows_main) % 8
    rows_last = n - (nw - 1) * rows_main
    assert rows_last > 0

    def chunk_list(total):
        out, off = [], 0
        while off < total:
            out.append((off, min(_CHUNK, total - off)))
            off += _CHUNK
        return out

    mesh = plsc.VectorSubcoreMesh(core_axis_name="c", subcore_axis_name="s")

    @functools.partial(
        pl.kernel,
        mesh=mesh,
        out_type=[
            jax.ShapeDtypeStruct((n_u, e), embed_user.dtype),
            jax.ShapeDtypeStruct((n_i, e), embed_item.dtype),
        ],
        scratch_types=[pltpu.VMEM((_CHUNK, 128), jnp.float32)],
    )
    def sc_copy(u_hbm, i_hbm, ou_hbm, oi_hbm, buf):
        wid = lax.axis_index("s") * info.num_cores + lax.axis_index("c")
        base = pl.multiple_of(wid * rows_main, 8)

        def do_copy(rows):
            for src, dst in ((u_hbm, ou_hbm), (i_hbm, oi_hbm)):
                for off, sz in chunk_list(rows):
                    pltpu.sync_copy(src.at[pl.ds(base + off, sz)],
                                    buf.at[pl.ds(0, sz)])
                    pltpu.sync_copy(buf.at[pl.ds(0, sz)],
                                    dst.at[pl.ds(base + off, sz)])

        @pl.when(wid < nw - 1)
        def _main():
            do_copy(rows_main)

        @pl.when(wid == nw - 1)
        def _tail():
            do_copy(rows_last)

    out_u, out_i = sc_copy(embed_user, embed_item)
    return (out_u, out_i)


# speedup vs baseline: 1.2251x; 1.2251x over previous
import functools

import jax
import jax.numpy as jnp
from jax import lax
from jax.experimental import pallas as pl
from jax.experimental.pallas import tpu as pltpu
from jax.experimental.pallas import tpu_sc as plsc

_TC_BLOCK_ROWS = 25000
_SC_CHUNK = 256


def _tc_copy(embed_user):
    n, e = embed_user.shape
    grid = (-(-n // _TC_BLOCK_ROWS),)
    spec = pl.BlockSpec((_TC_BLOCK_ROWS, e), lambda i: (i, 0))
    return pl.pallas_call(
        lambda u_ref, o_ref: o_ref.__setitem__(..., u_ref[...]),
        grid=grid,
        in_specs=[spec],
        out_specs=spec,
        out_shape=jax.ShapeDtypeStruct((n, e), embed_user.dtype),
    )(embed_user)


def _sc_copy(embed_item):
    n, e = embed_item.shape
    info = plsc.get_sparse_core_info()
    nw = info.num_cores * info.num_subcores
    rows_main = -(-n // nw)
    rows_main += (-rows_main) % 8
    rows_last = n - (nw - 1) * rows_main
    assert rows_last > 0
    mesh = plsc.VectorSubcoreMesh(core_axis_name="c", subcore_axis_name="s")

    def chunk_list(total):
        out, off = [], 0
        while off < total:
            out.append((off, min(_SC_CHUNK, total - off)))
            off += _SC_CHUNK
        return out

    @functools.partial(
        pl.kernel,
        mesh=mesh,
        out_type=jax.ShapeDtypeStruct((n, e), embed_item.dtype),
        scratch_types=[
            pltpu.VMEM((2, _SC_CHUNK, 128), jnp.float32),
            pltpu.SemaphoreType.DMA((2,)),
            pltpu.SemaphoreType.DMA((2,)),
        ],
    )
    def sc_copy(i_hbm, oi_hbm, buf, sem_in, sem_out):
        wid = lax.axis_index("s") * info.num_cores + lax.axis_index("c")
        base = pl.multiple_of(wid * rows_main, 8)

        def pipelined_copy(rows):
            chunks = chunk_list(rows)
            n_c = len(chunks)

            def fill(c):
                off, sz = chunks[c]
                return pltpu.make_async_copy(
                    i_hbm.at[pl.ds(base + off, sz)],
                    buf.at[c % 2, pl.ds(0, sz)], sem_in.at[c % 2])

            def drain(c):
                off, sz = chunks[c]
                return pltpu.make_async_copy(
                    buf.at[c % 2, pl.ds(0, sz)],
                    oi_hbm.at[pl.ds(base + off, sz)], sem_out.at[c % 2])

            fill(0).start()
            for c in range(n_c):
                fill(c).wait()
                if c + 1 < n_c:
                    if c >= 1:
                        drain(c - 1).wait()
                    fill(c + 1).start()
                drain(c).start()
            for c in range(max(0, n_c - 2), n_c):
                drain(c).wait()

        @pl.when(wid < nw - 1)
        def _main():
            pipelined_copy(rows_main)

        @pl.when(wid == nw - 1)
        def _tail():
            pipelined_copy(rows_last)

    return sc_copy(embed_item)


def kernel(embed_user, embed_item):
    return (_tc_copy(embed_user), _sc_copy(embed_item))
